# baseline (speedup 1.0000x reference)
import jax
import jax.numpy as jnp
from jax import lax
from jax.experimental import pallas as pl
from jax.experimental.pallas import tpu as pltpu
from jax.experimental.pallas import tpu_sc as plsc

NW = 32
PAIRS = 98304
PER_W = PAIRS // NW
CHUNK = 128
NCH = PER_W // CHUNK
EMB = 64
N_POS = 16384
DICT_HALF = 500000


CONV_C = 4096
CONV_H = CONV_C // 2
CONV_GRID = (1000000 + CONV_C - 1) // CONV_C
TBL_ROWS = CONV_GRID * CONV_H


def _tc_conv_body(tt_ref, out_ref):
    blk = tt_ref[...]
    eye = jnp.float32(1.0) * (
        lax.broadcasted_iota(jnp.int32, (64, 64), 0)
        == lax.broadcasted_iota(jnp.int32, (64, 64), 1))
    t = jax.lax.dot_general(blk, eye, (((0,), (0,)), ((), ())),
                            preferred_element_type=jnp.float32)
    out_ref[...] = jnp.concatenate([t[:CONV_H, :], t[CONV_H:, :]], axis=1)


_tc_conv = pl.pallas_call(
    _tc_conv_body,
    grid=(CONV_GRID,),
    in_specs=[pl.BlockSpec((64, CONV_C), lambda c: (0, c))],
    out_specs=pl.BlockSpec((CONV_H, 128), lambda c: (c, 0)),
    out_shape=jax.ShapeDtypeStruct((TBL_ROWS, 128), jnp.float32),
)


def _sc_dots_body(hi_hbm, ho_hbm, pi_hbm, po_hbm, in_hbm, out_hbm, part_hbm,
                  hi_v, ho_v, pi_v, po_v, rin_v, rout_v, part_v, sem_i, sem_o):
    wid = lax.axis_index("s") * 2 + lax.axis_index("c")
    pltpu.sync_copy(hi_hbm.at[wid], hi_v)
    pltpu.sync_copy(ho_hbm.at[wid], ho_v)
    pltpu.sync_copy(pi_hbm.at[wid], pi_v)
    pltpu.sync_copy(po_hbm.at[wid], po_v)

    def chunk_body(c, carry):
        cp_i = pltpu.async_copy(in_hbm.at[hi_v.at[c]], rin_v, sem_i)
        cp_o = pltpu.async_copy(out_hbm.at[ho_v.at[c]], rout_v, sem_o)
        cp_i.wait()
        cp_o.wait()

        def group_body(g, carry2):
            offi = pi_v[c, pl.ds(g * 16, 16)]
            offo = po_v[c, pl.ds(g * 16, 16)]
            for l in range(16):
                p = g * 16 + l
                a = offi[l]
                b = offo[l]
                acc = rin_v[p, pl.ds(a, 16)] * rout_v[p, pl.ds(b, 16)]
                for k in range(1, EMB // 16):
                    acc = acc + (rin_v[p, pl.ds(a + 16 * k, 16)]
                                 * rout_v[p, pl.ds(b + 16 * k, 16)])
                part_v[g, pl.ds(l * 16, 16)] = acc
            return carry2

        lax.fori_loop(0, CHUNK // 16, group_body, 0)
        row0 = pl.multiple_of((wid * PER_W + c * CHUNK) // 16, 8)
        pltpu.sync_copy(part_v, part_hbm.at[pl.ds(row0, CHUNK // 16)])
        return carry

    lax.fori_loop(0, NCH, chunk_body, 0)


_sc_dots = pl.kernel(
    _sc_dots_body,
    mesh=plsc.VectorSubcoreMesh(core_axis_name="c", subcore_axis_name="s"),
    out_type=jax.ShapeDtypeStruct((PAIRS // 16, 256), jnp.float32),
    scratch_types=[
        pltpu.VMEM((NCH, CHUNK), jnp.int32),
        pltpu.VMEM((NCH, CHUNK), jnp.int32),
        pltpu.VMEM((NCH, CHUNK), jnp.int32),
        pltpu.VMEM((NCH, CHUNK), jnp.int32),
        pltpu.VMEM((CHUNK, 128), jnp.float32),
        pltpu.VMEM((CHUNK, 128), jnp.float32),
        pltpu.VMEM((CHUNK // 16, 256), jnp.float32),
        pltpu.SemaphoreType.DMA,
        pltpu.SemaphoreType.DMA,
    ],
)


def _tc_loss_body(part_ref, out_ref):
    x = part_ref[...]
    col = lax.broadcasted_iota(jnp.int32, (256, 16), 0)
    grp = lax.broadcasted_iota(jnp.int32, (256, 16), 1)
    g = jnp.where(col // 16 == grp, 1.0, 0.0)
    s = jnp.dot(x, g, preferred_element_type=jnp.float32)
    row = lax.broadcasted_iota(jnp.int32, s.shape, 0)
    sign = jnp.where(row < N_POS // 16, 1.0, -1.0)
    y = sign * s
    ls = jnp.minimum(y, 0.0) - jnp.log(1.0 + jnp.exp(-jnp.abs(y)))
    out_ref[...] = jnp.full((1, 1), -jnp.sum(ls), jnp.float32)


_tc_loss = pl.pallas_call(
    _tc_loss_body,
    out_shape=jax.ShapeDtypeStruct((1, 1), jnp.float32),
)


def kernel(pos_in, pos_out, neg_in, neg_out, in_emb, out_emb):
    ii = jnp.concatenate([pos_in, neg_in]).astype(jnp.int32)
    oi = jnp.concatenate([pos_out, neg_out]).astype(jnp.int32)
    hi = (ii >> 1).reshape(NW, NCH, CHUNK)
    pi = ((ii & 1) << 6).reshape(NW, NCH, CHUNK)
    ho = ((oi // CONV_C) * CONV_H + (oi % CONV_C) % CONV_H).reshape(NW, NCH, CHUNK)
    po = (((oi % CONV_C) // CONV_H) << 6).reshape(NW, NCH, CHUNK)
    in2 = in_emb.reshape(DICT_HALF, 128)
    out2 = _tc_conv(out_emb.T)
    part = _sc_dots(hi, ho, pi, po, in2, out2)
    loss = _tc_loss(part)
    return loss[0, 0]


# speedup vs baseline: 1.1826x; 1.1826x over previous
import jax
import jax.numpy as jnp
from jax import lax
from jax.experimental import pallas as pl
from jax.experimental.pallas import tpu as pltpu
from jax.experimental.pallas import tpu_sc as plsc

NW = 32
PAIRS = 98304
PER_W = PAIRS // NW
CHUNK = 128
NCH = PER_W // CHUNK
EMB = 64
N_POS = 16384
DICT_HALF = 500000


CONV_C = 4096
CONV_H = CONV_C // 2
CONV_GRID = (1000000 + CONV_C - 1) // CONV_C
TBL_ROWS = CONV_GRID * CONV_H


def _tc_conv_body(tt_ref, out_ref):
    blk = tt_ref[...]
    eye = jnp.float32(1.0) * (
        lax.broadcasted_iota(jnp.int32, (64, 64), 0)
        == lax.broadcasted_iota(jnp.int32, (64, 64), 1))
    t = jax.lax.dot_general(blk, eye, (((0,), (0,)), ((), ())),
                            preferred_element_type=jnp.float32)
    out_ref[...] = jnp.concatenate([t[:CONV_H, :], t[CONV_H:, :]], axis=1)


_tc_conv = pl.pallas_call(
    _tc_conv_body,
    grid=(CONV_GRID,),
    in_specs=[pl.BlockSpec((64, CONV_C), lambda c: (0, c))],
    out_specs=pl.BlockSpec((CONV_H, 128), lambda c: (c, 0)),
    out_shape=jax.ShapeDtypeStruct((TBL_ROWS, 128), jnp.float32),
)


def _sc_dots_body(hi_hbm, ho_hbm, pi_hbm, po_hbm, in_hbm, out_hbm, part_hbm,
                  hi_v, ho_v, pi_v, po_v, rin_v, rout_v, part_v, sem_i, sem_o):
    wid = lax.axis_index("s") * 2 + lax.axis_index("c")
    pltpu.sync_copy(hi_hbm.at[wid], hi_v)
    pltpu.sync_copy(ho_hbm.at[wid], ho_v)
    pltpu.sync_copy(pi_hbm.at[wid], pi_v)
    pltpu.sync_copy(po_hbm.at[wid], po_v)

    def chunk_body(c, carry):
        cp_i = pltpu.async_copy(in_hbm.at[hi_v.at[c]], rin_v, sem_i)
        cp_o = pltpu.async_copy(out_hbm.at[ho_v.at[c]], rout_v, sem_o)
        cp_i.wait()
        cp_o.wait()

        def group_body(g, carry2):
            offi = pi_v[c, pl.ds(g * 16, 16)]
            offo = po_v[c, pl.ds(g * 16, 16)]
            for l in range(16):
                p = g * 16 + l
                a = offi[l]
                b = offo[l]
                acc = rin_v[p, pl.ds(a, 16)] * rout_v[p, pl.ds(b, 16)]
                for k in range(1, EMB // 16):
                    acc = acc + (rin_v[p, pl.ds(a + 16 * k, 16)]
                                 * rout_v[p, pl.ds(b + 16 * k, 16)])
                part_v[g, pl.ds(l * 16, 16)] = acc
            return carry2

        lax.fori_loop(0, CHUNK // 16, group_body, 0)
        row0 = pl.multiple_of((wid * PER_W + c * CHUNK) // 16, 8)
        pltpu.sync_copy(part_v, part_hbm.at[pl.ds(row0, CHUNK // 16)])
        return carry

    lax.fori_loop(0, NCH, chunk_body, 0)


_sc_dots = pl.kernel(
    _sc_dots_body,
    mesh=plsc.VectorSubcoreMesh(core_axis_name="c", subcore_axis_name="s"),
    out_type=jax.ShapeDtypeStruct((PAIRS // 16, 256), jnp.float32),
    scratch_types=[
        pltpu.VMEM((NCH, CHUNK), jnp.int32),
        pltpu.VMEM((NCH, CHUNK), jnp.int32),
        pltpu.VMEM((NCH, CHUNK), jnp.int32),
        pltpu.VMEM((NCH, CHUNK), jnp.int32),
        pltpu.VMEM((CHUNK, 128), jnp.float32),
        pltpu.VMEM((CHUNK, 128), jnp.float32),
        pltpu.VMEM((CHUNK // 16, 256), jnp.float32),
        pltpu.SemaphoreType.DMA,
        pltpu.SemaphoreType.DMA,
    ],
)


def _tc_loss_body(part_ref, out_ref):
    x = part_ref[...]
    col = lax.broadcasted_iota(jnp.int32, (256, 16), 0)
    grp = lax.broadcasted_iota(jnp.int32, (256, 16), 1)
    g = jnp.where(col // 16 == grp, 1.0, 0.0)
    s = jnp.dot(x, g, preferred_element_type=jnp.float32)
    row = lax.broadcasted_iota(jnp.int32, s.shape, 0)
    sign = jnp.where(row < N_POS // 16, 1.0, -1.0)
    y = sign * s
    ls = jnp.minimum(y, 0.0) - jnp.log(1.0 + jnp.exp(-jnp.abs(y)))
    out_ref[...] = jnp.full((1, 1), -jnp.sum(ls), jnp.float32)


_tc_loss = pl.pallas_call(
    _tc_loss_body,
    out_shape=jax.ShapeDtypeStruct((1, 1), jnp.float32),
)


def kernel(pos_in, pos_out, neg_in, neg_out, in_emb, out_emb):
    ii = jnp.concatenate([pos_in, neg_in]).astype(jnp.int32)
    oi = jnp.concatenate([pos_out, neg_out]).astype(jnp.int32)
    hi = ((ii // CONV_C) * CONV_H + (ii % CONV_C) % CONV_H).reshape(NW, NCH, CHUNK)
    ho = ((oi // CONV_C) * CONV_H + (oi % CONV_C) % CONV_H).reshape(NW, NCH, CHUNK)
    pi = (((ii % CONV_C) // CONV_H) << 6).reshape(NW, NCH, CHUNK)
    po = (((oi % CONV_C) // CONV_H) << 6).reshape(NW, NCH, CHUNK)
    in2 = _tc_conv(in_emb.T)
    out2 = _tc_conv(out_emb.T)
    part = _sc_dots(hi, ho, pi, po, in2, out2)
    loss = _tc_loss(part)
    return loss[0, 0]


# speedup vs baseline: 1.4347x; 1.2132x over previous
import jax
import jax.numpy as jnp
from jax import lax
from jax.experimental import pallas as pl
from jax.experimental.pallas import tpu as pltpu
from jax.experimental.pallas import tpu_sc as plsc

NW = 32
PAIRS = 98304
PER_W = PAIRS // NW
CHUNK = 128
NCH = PER_W // CHUNK
EMB = 64
N_POS = 16384
DICT_HALF = 500000


CONV_C = 8192
CONV_H = CONV_C // 2
CONV_GRID = (1000000 + CONV_C - 1) // CONV_C
TBL_ROWS = CONV_GRID * CONV_H


def _tc_conv_body(tt_ref, out_ref):
    blk = tt_ref[...]
    eye = jnp.float32(1.0) * (
        lax.broadcasted_iota(jnp.int32, (64, 64), 0)
        == lax.broadcasted_iota(jnp.int32, (64, 64), 1))
    t = jax.lax.dot_general(blk, eye, (((0,), (0,)), ((), ())),
                            preferred_element_type=jnp.float32)
    out_ref[...] = jnp.concatenate([t[:CONV_H, :], t[CONV_H:, :]], axis=1)


_tc_conv = pl.pallas_call(
    _tc_conv_body,
    grid=(CONV_GRID,),
    in_specs=[pl.BlockSpec((64, CONV_C), lambda c: (0, c))],
    out_specs=pl.BlockSpec((CONV_H, 128), lambda c: (c, 0)),
    out_shape=jax.ShapeDtypeStruct((TBL_ROWS, 128), jnp.float32),
)


def _sc_dots_body(hi_hbm, ho_hbm, pi_hbm, po_hbm, in_hbm, out_hbm, part_hbm,
                  hi_v, ho_v, pi_v, po_v, rin_v, rout_v, part_v, sem_i, sem_o):
    wid = lax.axis_index("s") * 2 + lax.axis_index("c")
    pltpu.sync_copy(hi_hbm.at[wid], hi_v)
    pltpu.sync_copy(ho_hbm.at[wid], ho_v)
    pltpu.sync_copy(pi_hbm.at[wid], pi_v)
    pltpu.sync_copy(po_hbm.at[wid], po_v)

    def chunk_body(c, carry):
        cp_i = pltpu.async_copy(in_hbm.at[hi_v.at[c]], rin_v, sem_i)
        cp_o = pltpu.async_copy(out_hbm.at[ho_v.at[c]], rout_v, sem_o)
        cp_i.wait()
        cp_o.wait()

        def group_body(g, carry2):
            offi = pi_v[c, pl.ds(g * 16, 16)]
            offo = po_v[c, pl.ds(g * 16, 16)]
            for l in range(16):
                p = g * 16 + l
                a = offi[l]
                b = offo[l]
                acc = rin_v[p, pl.ds(a, 16)] * rout_v[p, pl.ds(b, 16)]
                for k in range(1, EMB // 16):
                    acc = acc + (rin_v[p, pl.ds(a + 16 * k, 16)]
                                 * rout_v[p, pl.ds(b + 16 * k, 16)])
                part_v[g, pl.ds(l * 16, 16)] = acc
            return carry2

        lax.fori_loop(0, CHUNK // 16, group_body, 0)
        row0 = pl.multiple_of((wid * PER_W + c * CHUNK) // 16, 8)
        pltpu.sync_copy(part_v, part_hbm.at[pl.ds(row0, CHUNK // 16)])
        return carry

    lax.fori_loop(0, NCH, chunk_body, 0)


_sc_dots = pl.kernel(
    _sc_dots_body,
    mesh=plsc.VectorSubcoreMesh(core_axis_name="c", subcore_axis_name="s"),
    out_type=jax.ShapeDtypeStruct((PAIRS // 16, 256), jnp.float32),
    scratch_types=[
        pltpu.VMEM((NCH, CHUNK), jnp.int32),
        pltpu.VMEM((NCH, CHUNK), jnp.int32),
        pltpu.VMEM((NCH, CHUNK), jnp.int32),
        pltpu.VMEM((NCH, CHUNK), jnp.int32),
        pltpu.VMEM((CHUNK, 128), jnp.float32),
        pltpu.VMEM((CHUNK, 128), jnp.float32),
        pltpu.VMEM((CHUNK // 16, 256), jnp.float32),
        pltpu.SemaphoreType.DMA,
        pltpu.SemaphoreType.DMA,
    ],
)


def _tc_loss_body(part_ref, out_ref):
    x = part_ref[...]
    col = lax.broadcasted_iota(jnp.int32, (256, 16), 0)
    grp = lax.broadcasted_iota(jnp.int32, (256, 16), 1)
    g = jnp.where(col // 16 == grp, 1.0, 0.0)
    s = jnp.dot(x, g, preferred_element_type=jnp.float32)
    row = lax.broadcasted_iota(jnp.int32, s.shape, 0)
    sign = jnp.where(row < N_POS // 16, 1.0, -1.0)
    y = sign * s
    ls = jnp.minimum(y, 0.0) - jnp.log(1.0 + jnp.exp(-jnp.abs(y)))
    out_ref[...] = jnp.full((1, 1), -jnp.sum(ls), jnp.float32)


_tc_loss = pl.pallas_call(
    _tc_loss_body,
    out_shape=jax.ShapeDtypeStruct((1, 1), jnp.float32),
)


def kernel(pos_in, pos_out, neg_in, neg_out, in_emb, out_emb):
    ii = jnp.concatenate([pos_in, neg_in]).astype(jnp.int32)
    oi = jnp.concatenate([pos_out, neg_out]).astype(jnp.int32)
    hi = ((ii // CONV_C) * CONV_H + (ii % CONV_C) % CONV_H).reshape(NW, NCH, CHUNK)
    ho = ((oi // CONV_C) * CONV_H + (oi % CONV_C) % CONV_H).reshape(NW, NCH, CHUNK)
    pi = (((ii % CONV_C) // CONV_H) << 6).reshape(NW, NCH, CHUNK)
    po = (((oi % CONV_C) // CONV_H) << 6).reshape(NW, NCH, CHUNK)
    in2 = _tc_conv(in_emb.T)
    out2 = _tc_conv(out_emb.T)
    part = _sc_dots(hi, ho, pi, po, in2, out2)
    loss = _tc_loss(part)
    return loss[0, 0]


# speedup vs baseline: 1.6006x; 1.1156x over previous
import jax
import jax.numpy as jnp
from jax import lax
from jax.experimental import pallas as pl
from jax.experimental.pallas import tpu as pltpu
from jax.experimental.pallas import tpu_sc as plsc

NW = 32
PAIRS = 98304
PER_W = PAIRS // NW
CHUNK = 128
NCH = PER_W // CHUNK
EMB = 64
N_POS = 16384
DICT_HALF = 500000


CONV_C = 16384
CONV_H = CONV_C // 2
CONV_GRID = (1000000 + CONV_C - 1) // CONV_C
TBL_ROWS = CONV_GRID * CONV_H


def _tc_conv_body(tt_ref, out_ref):
    blk = tt_ref[...]
    eye = jnp.float32(1.0) * (
        lax.broadcasted_iota(jnp.int32, (64, 64), 0)
        == lax.broadcasted_iota(jnp.int32, (64, 64), 1))
    t = jax.lax.dot_general(blk, eye, (((0,), (0,)), ((), ())),
                            preferred_element_type=jnp.float32)
    out_ref[...] = jnp.concatenate([t[:CONV_H, :], t[CONV_H:, :]], axis=1)


_tc_conv = pl.pallas_call(
    _tc_conv_body,
    grid=(CONV_GRID,),
    in_specs=[pl.BlockSpec((64, CONV_C), lambda c: (0, c))],
    out_specs=pl.BlockSpec((CONV_H, 128), lambda c: (c, 0)),
    out_shape=jax.ShapeDtypeStruct((TBL_ROWS, 128), jnp.float32),
)


def _sc_dots_body(hi_hbm, ho_hbm, pi_hbm, po_hbm, in_hbm, out_hbm, part_hbm,
                  hi_v, ho_v, pi_v, po_v, rin_v, rout_v, part_v, sem_i, sem_o):
    wid = lax.axis_index("s") * 2 + lax.axis_index("c")
    pltpu.sync_copy(hi_hbm.at[wid], hi_v)
    pltpu.sync_copy(ho_hbm.at[wid], ho_v)
    pltpu.sync_copy(pi_hbm.at[wid], pi_v)
    pltpu.sync_copy(po_hbm.at[wid], po_v)

    def chunk_body(c, carry):
        cp_i = pltpu.async_copy(in_hbm.at[hi_v.at[c]], rin_v, sem_i)
        cp_o = pltpu.async_copy(out_hbm.at[ho_v.at[c]], rout_v, sem_o)
        cp_i.wait()
        cp_o.wait()

        def group_body(g, carry2):
            offi = pi_v[c, pl.ds(g * 16, 16)]
            offo = po_v[c, pl.ds(g * 16, 16)]
            for l in range(16):
                p = g * 16 + l
                a = offi[l]
                b = offo[l]
                acc = rin_v[p, pl.ds(a, 16)] * rout_v[p, pl.ds(b, 16)]
                for k in range(1, EMB // 16):
                    acc = acc + (rin_v[p, pl.ds(a + 16 * k, 16)]
                                 * rout_v[p, pl.ds(b + 16 * k, 16)])
                part_v[g, pl.ds(l * 16, 16)] = acc
            return carry2

        lax.fori_loop(0, CHUNK // 16, group_body, 0)
        row0 = pl.multiple_of((wid * PER_W + c * CHUNK) // 16, 8)
        pltpu.sync_copy(part_v, part_hbm.at[pl.ds(row0, CHUNK // 16)])
        return carry

    lax.fori_loop(0, NCH, chunk_body, 0)


_sc_dots = pl.kernel(
    _sc_dots_body,
    mesh=plsc.VectorSubcoreMesh(core_axis_name="c", subcore_axis_name="s"),
    out_type=jax.ShapeDtypeStruct((PAIRS // 16, 256), jnp.float32),
    scratch_types=[
        pltpu.VMEM((NCH, CHUNK), jnp.int32),
        pltpu.VMEM((NCH, CHUNK), jnp.int32),
        pltpu.VMEM((NCH, CHUNK), jnp.int32),
        pltpu.VMEM((NCH, CHUNK), jnp.int32),
        pltpu.VMEM((CHUNK, 128), jnp.float32),
        pltpu.VMEM((CHUNK, 128), jnp.float32),
        pltpu.VMEM((CHUNK // 16, 256), jnp.float32),
        pltpu.SemaphoreType.DMA,
        pltpu.SemaphoreType.DMA,
    ],
)


def _tc_loss_body(part_ref, out_ref):
    x = part_ref[...]
    col = lax.broadcasted_iota(jnp.int32, (256, 16), 0)
    grp = lax.broadcasted_iota(jnp.int32, (256, 16), 1)
    g = jnp.where(col // 16 == grp, 1.0, 0.0)
    s = jnp.dot(x, g, preferred_element_type=jnp.float32)
    row = lax.broadcasted_iota(jnp.int32, s.shape, 0)
    sign = jnp.where(row < N_POS // 16, 1.0, -1.0)
    y = sign * s
    ls = jnp.minimum(y, 0.0) - jnp.log(1.0 + jnp.exp(-jnp.abs(y)))
    out_ref[...] = jnp.full((1, 1), -jnp.sum(ls), jnp.float32)


_tc_loss = pl.pallas_call(
    _tc_loss_body,
    out_shape=jax.ShapeDtypeStruct((1, 1), jnp.float32),
)


def kernel(pos_in, pos_out, neg_in, neg_out, in_emb, out_emb):
    ii = jnp.concatenate([pos_in, neg_in]).astype(jnp.int32)
    oi = jnp.concatenate([pos_out, neg_out]).astype(jnp.int32)
    hi = ((ii // CONV_C) * CONV_H + (ii % CONV_C) % CONV_H).reshape(NW, NCH, CHUNK)
    ho = ((oi // CONV_C) * CONV_H + (oi % CONV_C) % CONV_H).reshape(NW, NCH, CHUNK)
    pi = (((ii % CONV_C) // CONV_H) << 6).reshape(NW, NCH, CHUNK)
    po = (((oi % CONV_C) // CONV_H) << 6).reshape(NW, NCH, CHUNK)
    in2 = _tc_conv(in_emb.T)
    out2 = _tc_conv(out_emb.T)
    part = _sc_dots(hi, ho, pi, po, in2, out2)
    loss = _tc_loss(part)
    return loss[0, 0]


# speedup vs baseline: 1.6846x; 1.0525x over previous
import jax
import jax.numpy as jnp
from jax import lax
from jax.experimental import pallas as pl
from jax.experimental.pallas import tpu as pltpu
from jax.experimental.pallas import tpu_sc as plsc

NW = 32
PAIRS = 98304
PER_W = PAIRS // NW
CHUNK = 128
NCH = PER_W // CHUNK
EMB = 64
N_POS = 16384
DICT_HALF = 500000


CONV_C = 32768
CONV_H = CONV_C // 2
CONV_GRID = (1000000 + CONV_C - 1) // CONV_C
TBL_ROWS = CONV_GRID * CONV_H


def _tc_conv_body(tt_ref, out_ref):
    blk = tt_ref[...]
    eye = jnp.float32(1.0) * (
        lax.broadcasted_iota(jnp.int32, (64, 64), 0)
        == lax.broadcasted_iota(jnp.int32, (64, 64), 1))
    t = jax.lax.dot_general(blk, eye, (((0,), (0,)), ((), ())),
                            preferred_element_type=jnp.float32)
    out_ref[...] = jnp.concatenate([t[:CONV_H, :], t[CONV_H:, :]], axis=1)


_tc_conv = pl.pallas_call(
    _tc_conv_body,
    grid=(CONV_GRID,),
    in_specs=[pl.BlockSpec((64, CONV_C), lambda c: (0, c))],
    out_specs=pl.BlockSpec((CONV_H, 128), lambda c: (c, 0)),
    out_shape=jax.ShapeDtypeStruct((TBL_ROWS, 128), jnp.float32),
)


def _sc_dots_body(hi_hbm, ho_hbm, pi_hbm, po_hbm, in_hbm, out_hbm, part_hbm,
                  hi_v, ho_v, pi_v, po_v, rin_v, rout_v, part_v, sem_i, sem_o):
    wid = lax.axis_index("s") * 2 + lax.axis_index("c")
    pltpu.sync_copy(hi_hbm.at[wid], hi_v)
    pltpu.sync_copy(ho_hbm.at[wid], ho_v)
    pltpu.sync_copy(pi_hbm.at[wid], pi_v)
    pltpu.sync_copy(po_hbm.at[wid], po_v)

    def chunk_body(c, carry):
        cp_i = pltpu.async_copy(in_hbm.at[hi_v.at[c]], rin_v, sem_i)
        cp_o = pltpu.async_copy(out_hbm.at[ho_v.at[c]], rout_v, sem_o)
        cp_i.wait()
        cp_o.wait()

        def group_body(g, carry2):
            offi = pi_v[c, pl.ds(g * 16, 16)]
            offo = po_v[c, pl.ds(g * 16, 16)]
            for l in range(16):
                p = g * 16 + l
                a = offi[l]
                b = offo[l]
                acc = rin_v[p, pl.ds(a, 16)] * rout_v[p, pl.ds(b, 16)]
                for k in range(1, EMB // 16):
                    acc = acc + (rin_v[p, pl.ds(a + 16 * k, 16)]
                                 * rout_v[p, pl.ds(b + 16 * k, 16)])
                part_v[g, pl.ds(l * 16, 16)] = acc
            return carry2

        lax.fori_loop(0, CHUNK // 16, group_body, 0)
        row0 = pl.multiple_of((wid * PER_W + c * CHUNK) // 16, 8)
        pltpu.sync_copy(part_v, part_hbm.at[pl.ds(row0, CHUNK // 16)])
        return carry

    lax.fori_loop(0, NCH, chunk_body, 0)


_sc_dots = pl.kernel(
    _sc_dots_body,
    mesh=plsc.VectorSubcoreMesh(core_axis_name="c", subcore_axis_name="s"),
    out_type=jax.ShapeDtypeStruct((PAIRS // 16, 256), jnp.float32),
    scratch_types=[
        pltpu.VMEM((NCH, CHUNK), jnp.int32),
        pltpu.VMEM((NCH, CHUNK), jnp.int32),
        pltpu.VMEM((NCH, CHUNK), jnp.int32),
        pltpu.VMEM((NCH, CHUNK), jnp.int32),
        pltpu.VMEM((CHUNK, 128), jnp.float32),
        pltpu.VMEM((CHUNK, 128), jnp.float32),
        pltpu.VMEM((CHUNK // 16, 256), jnp.float32),
        pltpu.SemaphoreType.DMA,
        pltpu.SemaphoreType.DMA,
    ],
)


def _tc_loss_body(part_ref, out_ref):
    x = part_ref[...]
    col = lax.broadcasted_iota(jnp.int32, (256, 16), 0)
    grp = lax.broadcasted_iota(jnp.int32, (256, 16), 1)
    g = jnp.where(col // 16 == grp, 1.0, 0.0)
    s = jnp.dot(x, g, preferred_element_type=jnp.float32)
    row = lax.broadcasted_iota(jnp.int32, s.shape, 0)
    sign = jnp.where(row < N_POS // 16, 1.0, -1.0)
    y = sign * s
    ls = jnp.minimum(y, 0.0) - jnp.log(1.0 + jnp.exp(-jnp.abs(y)))
    out_ref[...] = jnp.full((1, 1), -jnp.sum(ls), jnp.float32)


_tc_loss = pl.pallas_call(
    _tc_loss_body,
    out_shape=jax.ShapeDtypeStruct((1, 1), jnp.float32),
)


def kernel(pos_in, pos_out, neg_in, neg_out, in_emb, out_emb):
    ii = jnp.concatenate([pos_in, neg_in]).astype(jnp.int32)
    oi = jnp.concatenate([pos_out, neg_out]).astype(jnp.int32)
    hi = ((ii // CONV_C) * CONV_H + (ii % CONV_C) % CONV_H).reshape(NW, NCH, CHUNK)
    ho = ((oi // CONV_C) * CONV_H + (oi % CONV_C) % CONV_H).reshape(NW, NCH, CHUNK)
    pi = (((ii % CONV_C) // CONV_H) << 6).reshape(NW, NCH, CHUNK)
    po = (((oi % CONV_C) // CONV_H) << 6).reshape(NW, NCH, CHUNK)
    in2 = _tc_conv(in_emb.T)
    out2 = _tc_conv(out_emb.T)
    part = _sc_dots(hi, ho, pi, po, in2, out2)
    loss = _tc_loss(part)
    return loss[0, 0]


# speedup vs baseline: 1.7767x; 1.0546x over previous
import jax
import jax.numpy as jnp
from jax import lax
from jax.experimental import pallas as pl
from jax.experimental.pallas import tpu as pltpu
from jax.experimental.pallas import tpu_sc as plsc

NW = 32
PAIRS = 98304
PER_W = PAIRS // NW
CHUNK = 128
NCH = PER_W // CHUNK
EMB = 64
N_POS = 16384
DICT_HALF = 500000


CONV_C = 36864
CONV_H = CONV_C // 2
CONV_GRID = (1000000 + CONV_C - 1) // CONV_C
TBL_ROWS = CONV_GRID * CONV_H


def _tc_conv_body(tt_ref, out_ref):
    blk = tt_ref[...]
    eye = jnp.float32(1.0) * (
        lax.broadcasted_iota(jnp.int32, (64, 64), 0)
        == lax.broadcasted_iota(jnp.int32, (64, 64), 1))
    t = jax.lax.dot_general(blk, eye, (((0,), (0,)), ((), ())),
                            preferred_element_type=jnp.float32)
    out_ref[...] = jnp.concatenate([t[:CONV_H, :], t[CONV_H:, :]], axis=1)


_tc_conv = pl.pallas_call(
    _tc_conv_body,
    grid=(CONV_GRID,),
    in_specs=[pl.BlockSpec((64, CONV_C), lambda c: (0, c))],
    out_specs=pl.BlockSpec((CONV_H, 128), lambda c: (c, 0)),
    out_shape=jax.ShapeDtypeStruct((TBL_ROWS, 128), jnp.float32),
)


def _sc_dots_body(hi_hbm, ho_hbm, pi_hbm, po_hbm, in_hbm, out_hbm, part_hbm,
                  hi_v, ho_v, pi_v, po_v, rin0, rout0, rin1, rout1, part_v,
                  si0, so0, si1, so1):
    wid = lax.axis_index("s") * 2 + lax.axis_index("c")
    pltpu.sync_copy(hi_hbm.at[wid], hi_v)
    pltpu.sync_copy(ho_hbm.at[wid], ho_v)
    pltpu.sync_copy(pi_hbm.at[wid], pi_v)
    pltpu.sync_copy(po_hbm.at[wid], po_v)

    bufs = ((rin0, rout0, si0, so0), (rin1, rout1, si1, so1))

    def start(c, rin, rout, si, so):
        pltpu.async_copy(in_hbm.at[hi_v.at[c]], rin, si)
        pltpu.async_copy(out_hbm.at[ho_v.at[c]], rout, so)

    def compute(c, rin, rout, si, so):
        pltpu.make_async_copy(in_hbm.at[hi_v.at[c]], rin, si).wait()
        pltpu.make_async_copy(out_hbm.at[ho_v.at[c]], rout, so).wait()

        def group_body(g, carry2):
            offi = pi_v[c, pl.ds(g * 16, 16)]
            offo = po_v[c, pl.ds(g * 16, 16)]
            for l in range(16):
                p = g * 16 + l
                a = offi[l]
                b = offo[l]
                acc = rin[p, pl.ds(a, 16)] * rout[p, pl.ds(b, 16)]
                for k in range(1, EMB // 16):
                    acc = acc + (rin[p, pl.ds(a + 16 * k, 16)]
                                 * rout[p, pl.ds(b + 16 * k, 16)])
                part_v[g, pl.ds(l * 16, 16)] = acc
            return carry2

        lax.fori_loop(0, CHUNK // 16, group_body, 0)
        row0 = pl.multiple_of((wid * PER_W + c * CHUNK) // 16, 8)
        pltpu.sync_copy(part_v, part_hbm.at[pl.ds(row0, CHUNK // 16)])

    start(0, *bufs[0])
    start(1, *bufs[1])

    def pipe_body(cc, carry):
        c0 = cc * 2
        compute(c0, *bufs[0])

        @pl.when(cc < NCH // 2 - 1)
        def _():
            start(c0 + 2, *bufs[0])

        compute(c0 + 1, *bufs[1])

        @pl.when(cc < NCH // 2 - 1)
        def _():
            start(c0 + 3, *bufs[1])

        return carry

    lax.fori_loop(0, NCH // 2, pipe_body, 0)


_sc_dots = pl.kernel(
    _sc_dots_body,
    mesh=plsc.VectorSubcoreMesh(core_axis_name="c", subcore_axis_name="s"),
    out_type=jax.ShapeDtypeStruct((PAIRS // 16, 256), jnp.float32),
    scratch_types=[
        pltpu.VMEM((NCH, CHUNK), jnp.int32),
        pltpu.VMEM((NCH, CHUNK), jnp.int32),
        pltpu.VMEM((NCH, CHUNK), jnp.int32),
        pltpu.VMEM((NCH, CHUNK), jnp.int32),
        pltpu.VMEM((CHUNK, 128), jnp.float32),
        pltpu.VMEM((CHUNK, 128), jnp.float32),
        pltpu.VMEM((CHUNK, 128), jnp.float32),
        pltpu.VMEM((CHUNK, 128), jnp.float32),
        pltpu.VMEM((CHUNK // 16, 256), jnp.float32),
        pltpu.SemaphoreType.DMA,
        pltpu.SemaphoreType.DMA,
        pltpu.SemaphoreType.DMA,
        pltpu.SemaphoreType.DMA,
    ],
)


def _tc_loss_body(part_ref, out_ref):
    x = part_ref[...]
    col = lax.broadcasted_iota(jnp.int32, (256, 16), 0)
    grp = lax.broadcasted_iota(jnp.int32, (256, 16), 1)
    g = jnp.where(col // 16 == grp, 1.0, 0.0)
    s = jnp.dot(x, g, preferred_element_type=jnp.float32)
    row = lax.broadcasted_iota(jnp.int32, s.shape, 0)
    sign = jnp.where(row < N_POS // 16, 1.0, -1.0)
    y = sign * s
    ls = jnp.minimum(y, 0.0) - jnp.log(1.0 + jnp.exp(-jnp.abs(y)))
    out_ref[...] = jnp.full((1, 1), -jnp.sum(ls), jnp.float32)


_tc_loss = pl.pallas_call(
    _tc_loss_body,
    out_shape=jax.ShapeDtypeStruct((1, 1), jnp.float32),
)


def kernel(pos_in, pos_out, neg_in, neg_out, in_emb, out_emb):
    ii = jnp.concatenate([pos_in, neg_in]).astype(jnp.int32)
    oi = jnp.concatenate([pos_out, neg_out]).astype(jnp.int32)
    hi = ((ii // CONV_C) * CONV_H + (ii % CONV_C) % CONV_H).reshape(NW, NCH, CHUNK)
    ho = ((oi // CONV_C) * CONV_H + (oi % CONV_C) % CONV_H).reshape(NW, NCH, CHUNK)
    pi = (((ii % CONV_C) // CONV_H) << 6).reshape(NW, NCH, CHUNK)
    po = (((oi % CONV_C) // CONV_H) << 6).reshape(NW, NCH, CHUNK)
    in2 = _tc_conv(in_emb.T)
    out2 = _tc_conv(out_emb.T)
    part = _sc_dots(hi, ho, pi, po, in2, out2)
    loss = _tc_loss(part)
    return loss[0, 0]
